# baseline (speedup 1.0000x reference)
import functools

import jax
import jax.numpy as jnp
from jax import lax
from jax.experimental import pallas as pl
from jax.experimental.pallas import tpu as pltpu
from jax.experimental.pallas import tpu_sc as plsc

NC = 2
NS = 16
L = 16
NW = NC * NS

N = 10000
E = 320000
D = 128
NP = 10240
SLICE = NP // NS
EP = 327680
ET = EP // NW

KD = 128
NCHD = ET // KD

KR = 80
NCHR = ET // KR
KB = 32
NB = NCHR // KB
RING = 2
ROUNDS = KB // RING

KS = 128
NCHS = EP // NS // KS
SRING = 8
SROUNDS = NCHS // SRING

NCHF = ET // KD

_mesh = plsc.VectorSubcoreMesh(core_axis_name="c", subcore_axis_name="s")
_sc_params = pltpu.CompilerParams(needs_layout_passes=False)


def _wid():
    return lax.axis_index("s") * NC + lax.axis_index("c")


def _tile_slice():
    sid = lax.axis_index("s")
    return pl.ds(sid * SLICE, SLICE)


@functools.partial(
    pl.kernel,
    out_type=jax.ShapeDtypeStruct((NC, NP), jnp.float32),
    mesh=_mesh,
    compiler_params=_sc_params,
    name="deg_sc",
    scratch_types=[
        pltpu.VMEM((NCHD, KD), jnp.int32),
        pltpu.VMEM((KD,), jnp.float32),
        pltpu.SemaphoreType.DMA,
        pltpu.VMEM_SHARED((NP,), jnp.float32),
    ],
)
def _deg_kernel(dst_hbm, zeros_hbm, out_hbm, idx_v, ones_v, sem, acc_sh):
    cid = lax.axis_index("c")
    sl = _tile_slice()
    for l in range(KD // L):
        ones_v[pl.ds(l * L, L)] = jnp.ones((L,), jnp.float32)
    pltpu.sync_copy(zeros_hbm.at[sl], acc_sh.at[sl])
    pltpu.sync_copy(dst_hbm.at[_wid()], idx_v)
    plsc.subcore_barrier()

    def fire(j, carry):
        pltpu.async_copy(ones_v, acc_sh.at[idx_v.at[j]], sem, add=True)
        return carry

    lax.fori_loop(0, NCHD, fire, 0)

    def drain(j, carry):
        pltpu.make_async_copy(ones_v, acc_sh.at[idx_v.at[0]], sem).wait()
        return carry

    lax.fori_loop(0, NCHD, drain, 0)
    plsc.subcore_barrier()
    pltpu.sync_copy(acc_sh.at[sl], out_hbm.at[cid, sl])


@functools.partial(
    pl.kernel,
    out_type=jax.ShapeDtypeStruct((NC, NP, D), jnp.float32),
    mesh=_mesh,
    compiler_params=_sc_params,
    name="rowagg_sc",
    scratch_types=[
        pltpu.VMEM((KB, KR), jnp.int32),
        pltpu.VMEM((KB, KR), jnp.int32),
        pltpu.VMEM((RING * KR, D), jnp.float32),
        [pltpu.SemaphoreType.DMA] * RING,
        [pltpu.SemaphoreType.DMA] * RING,
        pltpu.VMEM_SHARED((NP, D), jnp.float32),
    ],
)
def _row_agg_kernel(src_hbm, dst_hbm, y_hbm, zeros_hbm, out_hbm,
                    sblk, dblk, rowsb, gsem, ssem, acc_sh):
    bufs = [rowsb.at[pl.ds(t * KR, KR)] for t in range(RING)]
    cid = lax.axis_index("c")
    sl = _tile_slice()
    pltpu.sync_copy(zeros_hbm.at[sl], acc_sh.at[sl])
    w = _wid()
    plsc.subcore_barrier()

    for b in range(NB):
        pltpu.sync_copy(src_hbm.at[w, pl.ds(b * KB, KB)], sblk)
        pltpu.sync_copy(dst_hbm.at[w, pl.ds(b * KB, KB)], dblk)

        for t in range(RING):
            pltpu.async_copy(y_hbm.at[sblk.at[t]], bufs[t], gsem[t])

        def round_body(r, carry):
            for t in range(RING):
                j = RING * r + t
                pltpu.make_async_copy(
                    y_hbm.at[sblk.at[j]], bufs[t], gsem[t]).wait()
                pltpu.async_copy(bufs[t], acc_sh.at[dblk.at[j]], ssem[t],
                                 add=True)

            @pl.when(r < ROUNDS - 1)
            def _():
                for t in range(RING):
                    j = RING * r + t
                    pltpu.make_async_copy(
                        bufs[t], acc_sh.at[dblk.at[j]], ssem[t]).wait()
                    pltpu.async_copy(
                        y_hbm.at[sblk.at[j + RING]], bufs[t], gsem[t])

            return carry

        lax.fori_loop(0, ROUNDS, round_body, 0)
        for t in range(RING):
            pltpu.make_async_copy(
                bufs[t], acc_sh.at[dblk.at[KB - RING + t]], ssem[t]).wait()

    plsc.subcore_barrier()
    pltpu.sync_copy(acc_sh.at[sl], out_hbm.at[cid, sl])


@functools.partial(
    pl.kernel,
    out_type=jax.ShapeDtypeStruct((NW, NCHF, KD), jnp.float32),
    mesh=_mesh,
    compiler_params=_sc_params,
    name="scaldec_sc",
    scratch_types=[
        pltpu.VMEM((NCHS, KS), jnp.int32),
        pltpu.VMEM((NCHS, KS), jnp.int32),
        pltpu.VMEM((NCHF, KD), jnp.int32),
        pltpu.VMEM((NCHF, KD), jnp.int32),
        pltpu.VMEM((NP,), jnp.float32),
        pltpu.VMEM((NP,), jnp.float32),
        pltpu.VMEM((NP,), jnp.float32),
        pltpu.VMEM((SRING, KS), jnp.float32),
        pltpu.VMEM((NCHF, KD), jnp.float32),
        pltpu.VMEM((L,), jnp.float32),
        [pltpu.SemaphoreType.DMA] * SRING,
        pltpu.VMEM_SHARED((NP,), jnp.float32),
    ],
)
def _scal_decode_kernel(src16_hbm, dst16_hbm, src32_hbm, dst32_hbm,
                        u_hbm, dis_hbm, zeros_hbm, b2_hbm, out_hbm,
                        sidx, didx, fsidx, fdidx, u_v, dis_v, z_v,
                        ring, out_v, b2_v, ssem, acc_sh):
    sid = lax.axis_index("s")
    sl = _tile_slice()
    pltpu.sync_copy(zeros_hbm.at[sl], acc_sh.at[sl])
    pltpu.sync_copy(src16_hbm.at[sid], sidx)
    pltpu.sync_copy(dst16_hbm.at[sid], didx)
    pltpu.sync_copy(u_hbm, u_v)
    pltpu.sync_copy(dis_hbm, dis_v)
    pltpu.sync_copy(b2_hbm, b2_v)
    plsc.subcore_barrier()

    def srow(r, carry):
        for t in range(SRING):
            j = SRING * r + t

            @pl.when(r > 0)
            def _():
                pltpu.make_async_copy(
                    ring.at[t], acc_sh.at[didx.at[j]], ssem[t]).wait()

            for l in range(KS // L):
                iv = sidx[j, pl.ds(l * L, L)]
                ring[t, pl.ds(l * L, L)] = plsc.load_gather(u_v, [iv])
            pltpu.async_copy(ring.at[t], acc_sh.at[didx.at[j]], ssem[t],
                             add=True)
        return carry

    lax.fori_loop(0, SROUNDS, srow, 0)
    for t in range(SRING):
        pltpu.make_async_copy(
            ring.at[t], acc_sh.at[didx.at[0]], ssem[t]).wait()
    plsc.subcore_barrier()

    pltpu.sync_copy(acc_sh, z_v)
    w = _wid()
    pltpu.sync_copy(src32_hbm.at[w], fsidx)
    pltpu.sync_copy(dst32_hbm.at[w], fdidx)
    b2 = b2_v[...]

    def zbody(i, carry):
        s = pl.ds(i * L, L)
        z_v[s] = dis_v[s] * (z_v[s] + u_v[s]) + b2
        return carry

    lax.fori_loop(0, NP // L, zbody, 0)

    one = jnp.ones((L,), jnp.float32)

    def dchunk(j, carry):
        for l in range(KD // L):
            cs = pl.ds(l * L, L)
            zs = plsc.load_gather(z_v, [fsidx[j, cs]])
            zd = plsc.load_gather(z_v, [fdidx[j, cs]])
            out_v[j, cs] = one / (one + jnp.exp(-(zs * zd)))
        return carry

    lax.fori_loop(0, NCHF, dchunk, 0)
    pltpu.sync_copy(out_v, out_hbm.at[w])


def _tc_encode_body(degp_ref, x_ref, w1_ref, y_ref, dis_ref):
    deg = degp_ref[0, :] + degp_ref[1, :] + 1.0
    dis = lax.rsqrt(deg)
    xw = jnp.dot(x_ref[...], w1_ref[...], preferred_element_type=jnp.float32)
    y_ref[...] = xw * dis[:, None]
    dis_ref[...] = dis


def _tc_mid_body(accp_ref, y_ref, dis_ref, b1_ref, w2_ref, u_ref):
    dis = dis_ref[...]
    t = (accp_ref[0] + accp_ref[1] + y_ref[...]) * dis[:, None]
    h = jnp.maximum(t + b1_ref[...][None, :], 0.0)
    hw = jnp.sum(h * w2_ref[...][:, 0][None, :], axis=1)
    u_ref[...] = hw * dis


def kernel(x, edge_index, W1, b1, W2, b2):
    base = jnp.arange(EP - E, dtype=jnp.int32)
    padA = N + base % (NP - N)
    padB = N + (base + 80) % (NP - N)
    padC = N + (base + 160) % (NP - N)
    srcA = jnp.concatenate([edge_index[0], padA])
    dstA = jnp.concatenate([edge_index[1], padA])
    srcB = jnp.concatenate([edge_index[0], padB])
    dstB = jnp.concatenate([edge_index[1], padB])
    srcC = jnp.concatenate([edge_index[0], padC])
    dstC = jnp.concatenate([edge_index[1], padC])
    dst_deg = dstA.reshape(NW, NCHD, KD)
    src_ra = srcB.reshape(NW, NCHR, KR)
    dst_ra = dstB.reshape(NW, NCHR, KR)
    src16 = srcC.reshape(NS, NCHS, KS)
    dst16 = dstC.reshape(NS, NCHS, KS)
    src_de = srcA.reshape(NW, NCHF, KD)
    dst_de = dstA.reshape(NW, NCHF, KD)

    xp = jnp.pad(x, ((0, NP - N), (0, 0)))
    zeros_n = jnp.zeros((NP,), jnp.float32)
    zeros_nd = jnp.zeros((NP, D), jnp.float32)
    b2v = jnp.broadcast_to(b2.astype(jnp.float32), (L,))

    degp = _deg_kernel(dst_deg, zeros_n)

    y, dis = pl.pallas_call(
        _tc_encode_body,
        out_shape=[
            jax.ShapeDtypeStruct((NP, D), jnp.float32),
            jax.ShapeDtypeStruct((NP,), jnp.float32),
        ],
    )(degp, xp, W1)

    accp = _row_agg_kernel(src_ra, dst_ra, y, zeros_nd)

    u = pl.pallas_call(
        _tc_mid_body,
        out_shape=jax.ShapeDtypeStruct((NP,), jnp.float32),
    )(accp, y, dis, b1, W2)

    out = _scal_decode_kernel(src16, dst16, src_de, dst_de, u, dis,
                              zeros_n, b2v)
    return out.reshape(EP)[:E]


# speedup vs baseline: 1.0837x; 1.0837x over previous
import functools

import jax
import jax.numpy as jnp
from jax import lax
from jax.experimental import pallas as pl
from jax.experimental.pallas import tpu as pltpu
from jax.experimental.pallas import tpu_sc as plsc

NC = 2
NS = 16
L = 16
NW = NC * NS

N = 10000
E = 320000
D = 128
NP = 10240
SLICE = NP // NS
EP = 327680
ET = EP // NW

KD = 128
NCHD = ET // KD

KR = 40
NCHR = ET // KR
KB = 32
NB = NCHR // KB
RING = 4
ROUNDS = KB // RING

KS = 128
NCHS = EP // NS // KS
SRING = 8
SROUNDS = NCHS // SRING

NCHF = ET // KD

_mesh = plsc.VectorSubcoreMesh(core_axis_name="c", subcore_axis_name="s")
_sc_params = pltpu.CompilerParams(needs_layout_passes=False)


def _wid():
    return lax.axis_index("s") * NC + lax.axis_index("c")


def _tile_slice():
    sid = lax.axis_index("s")
    return pl.ds(sid * SLICE, SLICE)


@functools.partial(
    pl.kernel,
    out_type=jax.ShapeDtypeStruct((NC, NP), jnp.float32),
    mesh=_mesh,
    compiler_params=_sc_params,
    name="deg_sc",
    scratch_types=[
        pltpu.VMEM((NCHD, KD), jnp.int32),
        pltpu.VMEM((KD,), jnp.float32),
        pltpu.SemaphoreType.DMA,
        pltpu.VMEM_SHARED((NP,), jnp.float32),
    ],
)
def _deg_kernel(dst_hbm, zeros_hbm, out_hbm, idx_v, ones_v, sem, acc_sh):
    cid = lax.axis_index("c")
    sl = _tile_slice()
    for l in range(KD // L):
        ones_v[pl.ds(l * L, L)] = jnp.ones((L,), jnp.float32)
    pltpu.sync_copy(zeros_hbm.at[sl], acc_sh.at[sl])
    pltpu.sync_copy(dst_hbm.at[_wid()], idx_v)
    plsc.subcore_barrier()

    def fire(j, carry):
        pltpu.async_copy(ones_v, acc_sh.at[idx_v.at[j]], sem, add=True)
        return carry

    lax.fori_loop(0, NCHD, fire, 0)

    def drain(j, carry):
        pltpu.make_async_copy(ones_v, acc_sh.at[idx_v.at[0]], sem).wait()
        return carry

    lax.fori_loop(0, NCHD, drain, 0)
    plsc.subcore_barrier()
    pltpu.sync_copy(acc_sh.at[sl], out_hbm.at[cid, sl])


@functools.partial(
    pl.kernel,
    out_type=jax.ShapeDtypeStruct((NC, NP, D), jnp.float32),
    mesh=_mesh,
    compiler_params=_sc_params,
    name="rowagg_sc",
    scratch_types=[
        pltpu.VMEM((KB, KR), jnp.int32),
        pltpu.VMEM((KB, KR), jnp.int32),
        pltpu.VMEM((RING * KR, D), jnp.float32),
        [pltpu.SemaphoreType.DMA] * RING,
        [pltpu.SemaphoreType.DMA] * RING,
        pltpu.VMEM_SHARED((NP, D), jnp.float32),
    ],
)
def _row_agg_kernel(src_hbm, dst_hbm, y_hbm, zeros_hbm, out_hbm,
                    sblk, dblk, rowsb, gsem, ssem, acc_sh):
    bufs = [rowsb.at[pl.ds(t * KR, KR)] for t in range(RING)]
    cid = lax.axis_index("c")
    sl = _tile_slice()
    pltpu.sync_copy(zeros_hbm.at[sl], acc_sh.at[sl])
    w = _wid()
    plsc.subcore_barrier()

    for b in range(NB):
        pltpu.sync_copy(src_hbm.at[w, pl.ds(b * KB, KB)], sblk)
        pltpu.sync_copy(dst_hbm.at[w, pl.ds(b * KB, KB)], dblk)

        for t in range(RING):
            pltpu.async_copy(y_hbm.at[sblk.at[t]], bufs[t], gsem[t])

        def round_body(r, carry):
            for t in range(RING):
                j = RING * r + t
                pltpu.make_async_copy(
                    y_hbm.at[sblk.at[j]], bufs[t], gsem[t]).wait()
                pltpu.async_copy(bufs[t], acc_sh.at[dblk.at[j]], ssem[t],
                                 add=True)

            @pl.when(r < ROUNDS - 1)
            def _():
                for t in range(RING):
                    j = RING * r + t
                    pltpu.make_async_copy(
                        bufs[t], acc_sh.at[dblk.at[j]], ssem[t]).wait()
                    pltpu.async_copy(
                        y_hbm.at[sblk.at[j + RING]], bufs[t], gsem[t])

            return carry

        lax.fori_loop(0, ROUNDS, round_body, 0)
        for t in range(RING):
            pltpu.make_async_copy(
                bufs[t], acc_sh.at[dblk.at[KB - RING + t]], ssem[t]).wait()

    plsc.subcore_barrier()
    pltpu.sync_copy(acc_sh.at[sl], out_hbm.at[cid, sl])


@functools.partial(
    pl.kernel,
    out_type=jax.ShapeDtypeStruct((NW, NCHF, KD), jnp.float32),
    mesh=_mesh,
    compiler_params=_sc_params,
    name="scaldec_sc",
    scratch_types=[
        pltpu.VMEM((NCHS, KS), jnp.int32),
        pltpu.VMEM((NCHS, KS), jnp.int32),
        pltpu.VMEM((NCHF, KD), jnp.int32),
        pltpu.VMEM((NCHF, KD), jnp.int32),
        pltpu.VMEM((NP,), jnp.float32),
        pltpu.VMEM((NP,), jnp.float32),
        pltpu.VMEM((NP,), jnp.float32),
        pltpu.VMEM((SRING, KS), jnp.float32),
        pltpu.VMEM((NCHF, KD), jnp.float32),
        pltpu.VMEM((L,), jnp.float32),
        [pltpu.SemaphoreType.DMA] * SRING,
        pltpu.VMEM_SHARED((NP,), jnp.float32),
    ],
)
def _scal_decode_kernel(src16_hbm, dst16_hbm, src32_hbm, dst32_hbm,
                        u_hbm, dis_hbm, zeros_hbm, b2_hbm, out_hbm,
                        sidx, didx, fsidx, fdidx, u_v, dis_v, z_v,
                        ring, out_v, b2_v, ssem, acc_sh):
    sid = lax.axis_index("s")
    sl = _tile_slice()
    pltpu.sync_copy(zeros_hbm.at[sl], acc_sh.at[sl])
    pltpu.sync_copy(src16_hbm.at[sid], sidx)
    pltpu.sync_copy(dst16_hbm.at[sid], didx)
    pltpu.sync_copy(u_hbm, u_v)
    pltpu.sync_copy(dis_hbm, dis_v)
    pltpu.sync_copy(b2_hbm, b2_v)
    plsc.subcore_barrier()

    def srow(r, carry):
        for t in range(SRING):
            j = SRING * r + t

            @pl.when(r > 0)
            def _():
                pltpu.make_async_copy(
                    ring.at[t], acc_sh.at[didx.at[j]], ssem[t]).wait()

            for l in range(KS // L):
                iv = sidx[j, pl.ds(l * L, L)]
                ring[t, pl.ds(l * L, L)] = plsc.load_gather(u_v, [iv])
            pltpu.async_copy(ring.at[t], acc_sh.at[didx.at[j]], ssem[t],
                             add=True)
        return carry

    lax.fori_loop(0, SROUNDS, srow, 0)
    for t in range(SRING):
        pltpu.make_async_copy(
            ring.at[t], acc_sh.at[didx.at[0]], ssem[t]).wait()
    plsc.subcore_barrier()

    pltpu.sync_copy(acc_sh, z_v)
    w = _wid()
    pltpu.sync_copy(src32_hbm.at[w], fsidx)
    pltpu.sync_copy(dst32_hbm.at[w], fdidx)
    b2 = b2_v[...]

    def zbody(i, carry):
        s = pl.ds(i * L, L)
        z_v[s] = dis_v[s] * (z_v[s] + u_v[s]) + b2
        return carry

    lax.fori_loop(0, NP // L, zbody, 0)

    one = jnp.ones((L,), jnp.float32)

    def dchunk(j, carry):
        for l in range(KD // L):
            cs = pl.ds(l * L, L)
            zs = plsc.load_gather(z_v, [fsidx[j, cs]])
            zd = plsc.load_gather(z_v, [fdidx[j, cs]])
            out_v[j, cs] = one / (one + jnp.exp(-(zs * zd)))
        return carry

    lax.fori_loop(0, NCHF, dchunk, 0)
    pltpu.sync_copy(out_v, out_hbm.at[w])


def _tc_encode_body(degp_ref, x_ref, w1_ref, y_ref, dis_ref):
    deg = degp_ref[0, :] + degp_ref[1, :] + 1.0
    dis = lax.rsqrt(deg)
    xw = jnp.dot(x_ref[...], w1_ref[...], preferred_element_type=jnp.float32)
    y_ref[...] = xw * dis[:, None]
    dis_ref[...] = dis


def _tc_mid_body(accp_ref, y_ref, dis_ref, b1_ref, w2_ref, u_ref):
    dis = dis_ref[...]
    t = (accp_ref[0] + accp_ref[1] + y_ref[...]) * dis[:, None]
    h = jnp.maximum(t + b1_ref[...][None, :], 0.0)
    hw = jnp.sum(h * w2_ref[...][:, 0][None, :], axis=1)
    u_ref[...] = hw * dis


def kernel(x, edge_index, W1, b1, W2, b2):
    base = jnp.arange(EP - E, dtype=jnp.int32)
    padA = N + base % (NP - N)
    padB = N + (base + 80) % (NP - N)
    padC = N + (base + 160) % (NP - N)
    srcA = jnp.concatenate([edge_index[0], padA])
    dstA = jnp.concatenate([edge_index[1], padA])
    srcB = jnp.concatenate([edge_index[0], padB])
    dstB = jnp.concatenate([edge_index[1], padB])
    srcC = jnp.concatenate([edge_index[0], padC])
    dstC = jnp.concatenate([edge_index[1], padC])
    dst_deg = dstA.reshape(NW, NCHD, KD)
    src_ra = srcB.reshape(NW, NCHR, KR)
    dst_ra = dstB.reshape(NW, NCHR, KR)
    src16 = srcC.reshape(NS, NCHS, KS)
    dst16 = dstC.reshape(NS, NCHS, KS)
    src_de = srcA.reshape(NW, NCHF, KD)
    dst_de = dstA.reshape(NW, NCHF, KD)

    xp = jnp.pad(x, ((0, NP - N), (0, 0)))
    zeros_n = jnp.zeros((NP,), jnp.float32)
    zeros_nd = jnp.zeros((NP, D), jnp.float32)
    b2v = jnp.broadcast_to(b2.astype(jnp.float32), (L,))

    degp = _deg_kernel(dst_deg, zeros_n)

    y, dis = pl.pallas_call(
        _tc_encode_body,
        out_shape=[
            jax.ShapeDtypeStruct((NP, D), jnp.float32),
            jax.ShapeDtypeStruct((NP,), jnp.float32),
        ],
    )(degp, xp, W1)

    accp = _row_agg_kernel(src_ra, dst_ra, y, zeros_nd)

    u = pl.pallas_call(
        _tc_mid_body,
        out_shape=jax.ShapeDtypeStruct((NP,), jnp.float32),
    )(accp, y, dis, b1, W2)

    out = _scal_decode_kernel(src16, dst16, src_de, dst_de, u, dis,
                              zeros_n, b2v)
    return out.reshape(EP)[:E]


# speedup vs baseline: 1.1097x; 1.0240x over previous
import functools

import jax
import jax.numpy as jnp
from jax import lax
from jax.experimental import pallas as pl
from jax.experimental.pallas import tpu as pltpu
from jax.experimental.pallas import tpu_sc as plsc

NC = 2
NS = 16
L = 16
NW = NC * NS

N = 10000
E = 320000
D = 128
NP = 10240
SLICE = NP // NS
EP = 327680
ET = EP // NW

KD = 128
NCHD = ET // KD

KR = 40
NCHR = ET // KR
KB = 32
NB = NCHR // KB
RING = 8
ROUNDS = KB // RING

KS = 128
NCHS = EP // NS // KS
SRING = 8
SROUNDS = NCHS // SRING

NCHF = ET // KD

_mesh = plsc.VectorSubcoreMesh(core_axis_name="c", subcore_axis_name="s")
_sc_params = pltpu.CompilerParams(needs_layout_passes=False)


def _wid():
    return lax.axis_index("s") * NC + lax.axis_index("c")


def _tile_slice():
    sid = lax.axis_index("s")
    return pl.ds(sid * SLICE, SLICE)


@functools.partial(
    pl.kernel,
    out_type=jax.ShapeDtypeStruct((NC, NP), jnp.float32),
    mesh=_mesh,
    compiler_params=_sc_params,
    name="deg_sc",
    scratch_types=[
        pltpu.VMEM((NCHD, KD), jnp.int32),
        pltpu.VMEM((KD,), jnp.float32),
        pltpu.SemaphoreType.DMA,
        pltpu.VMEM_SHARED((NP,), jnp.float32),
    ],
)
def _deg_kernel(dst_hbm, zeros_hbm, out_hbm, idx_v, ones_v, sem, acc_sh):
    cid = lax.axis_index("c")
    sl = _tile_slice()
    for l in range(KD // L):
        ones_v[pl.ds(l * L, L)] = jnp.ones((L,), jnp.float32)
    pltpu.sync_copy(zeros_hbm.at[sl], acc_sh.at[sl])
    pltpu.sync_copy(dst_hbm.at[_wid()], idx_v)
    plsc.subcore_barrier()

    def fire(j, carry):
        pltpu.async_copy(ones_v, acc_sh.at[idx_v.at[j]], sem, add=True)
        return carry

    lax.fori_loop(0, NCHD, fire, 0)

    def drain(j, carry):
        pltpu.make_async_copy(ones_v, acc_sh.at[idx_v.at[0]], sem).wait()
        return carry

    lax.fori_loop(0, NCHD, drain, 0)
    plsc.subcore_barrier()
    pltpu.sync_copy(acc_sh.at[sl], out_hbm.at[cid, sl])


@functools.partial(
    pl.kernel,
    out_type=jax.ShapeDtypeStruct((NC, NP, D), jnp.float32),
    mesh=_mesh,
    compiler_params=_sc_params,
    name="rowagg_sc",
    scratch_types=[
        pltpu.VMEM((KB, KR), jnp.int32),
        pltpu.VMEM((KB, KR), jnp.int32),
        pltpu.VMEM((RING * KR, D), jnp.float32),
        [pltpu.SemaphoreType.DMA] * RING,
        [pltpu.SemaphoreType.DMA] * RING,
        pltpu.VMEM_SHARED((NP, D), jnp.float32),
    ],
)
def _row_agg_kernel(src_hbm, dst_hbm, y_hbm, zeros_hbm, out_hbm,
                    sblk, dblk, rowsb, gsem, ssem, acc_sh):
    bufs = [rowsb.at[pl.ds(t * KR, KR)] for t in range(RING)]
    cid = lax.axis_index("c")
    sl = _tile_slice()
    pltpu.sync_copy(zeros_hbm.at[sl], acc_sh.at[sl])
    w = _wid()
    plsc.subcore_barrier()

    for b in range(NB):
        pltpu.sync_copy(src_hbm.at[w, pl.ds(b * KB, KB)], sblk)
        pltpu.sync_copy(dst_hbm.at[w, pl.ds(b * KB, KB)], dblk)

        for t in range(RING):
            pltpu.async_copy(y_hbm.at[sblk.at[t]], bufs[t], gsem[t])

        def round_body(r, carry):
            for t in range(RING):
                j = RING * r + t
                pltpu.make_async_copy(
                    y_hbm.at[sblk.at[j]], bufs[t], gsem[t]).wait()
                pltpu.async_copy(bufs[t], acc_sh.at[dblk.at[j]], ssem[t],
                                 add=True)

            @pl.when(r < ROUNDS - 1)
            def _():
                for t in range(RING):
                    j = RING * r + t
                    pltpu.make_async_copy(
                        bufs[t], acc_sh.at[dblk.at[j]], ssem[t]).wait()
                    pltpu.async_copy(
                        y_hbm.at[sblk.at[j + RING]], bufs[t], gsem[t])

            return carry

        lax.fori_loop(0, ROUNDS, round_body, 0)
        for t in range(RING):
            pltpu.make_async_copy(
                bufs[t], acc_sh.at[dblk.at[KB - RING + t]], ssem[t]).wait()

    plsc.subcore_barrier()
    pltpu.sync_copy(acc_sh.at[sl], out_hbm.at[cid, sl])


@functools.partial(
    pl.kernel,
    out_type=jax.ShapeDtypeStruct((NW, NCHF, KD), jnp.float32),
    mesh=_mesh,
    compiler_params=_sc_params,
    name="scaldec_sc",
    scratch_types=[
        pltpu.VMEM((NCHS, KS), jnp.int32),
        pltpu.VMEM((NCHS, KS), jnp.int32),
        pltpu.VMEM((NCHF, KD), jnp.int32),
        pltpu.VMEM((NCHF, KD), jnp.int32),
        pltpu.VMEM((NP,), jnp.float32),
        pltpu.VMEM((NP,), jnp.float32),
        pltpu.VMEM((NP,), jnp.float32),
        pltpu.VMEM((SRING, KS), jnp.float32),
        pltpu.VMEM((NCHF, KD), jnp.float32),
        pltpu.VMEM((L,), jnp.float32),
        [pltpu.SemaphoreType.DMA] * SRING,
        pltpu.VMEM_SHARED((NP,), jnp.float32),
    ],
)
def _scal_decode_kernel(src16_hbm, dst16_hbm, src32_hbm, dst32_hbm,
                        u_hbm, dis_hbm, zeros_hbm, b2_hbm, out_hbm,
                        sidx, didx, fsidx, fdidx, u_v, dis_v, z_v,
                        ring, out_v, b2_v, ssem, acc_sh):
    sid = lax.axis_index("s")
    sl = _tile_slice()
    pltpu.sync_copy(zeros_hbm.at[sl], acc_sh.at[sl])
    pltpu.sync_copy(src16_hbm.at[sid], sidx)
    pltpu.sync_copy(dst16_hbm.at[sid], didx)
    pltpu.sync_copy(u_hbm, u_v)
    pltpu.sync_copy(dis_hbm, dis_v)
    pltpu.sync_copy(b2_hbm, b2_v)
    plsc.subcore_barrier()

    def srow(r, carry):
        for t in range(SRING):
            j = SRING * r + t

            @pl.when(r > 0)
            def _():
                pltpu.make_async_copy(
                    ring.at[t], acc_sh.at[didx.at[j]], ssem[t]).wait()

            for l in range(KS // L):
                iv = sidx[j, pl.ds(l * L, L)]
                ring[t, pl.ds(l * L, L)] = plsc.load_gather(u_v, [iv])
            pltpu.async_copy(ring.at[t], acc_sh.at[didx.at[j]], ssem[t],
                             add=True)
        return carry

    lax.fori_loop(0, SROUNDS, srow, 0)
    for t in range(SRING):
        pltpu.make_async_copy(
            ring.at[t], acc_sh.at[didx.at[0]], ssem[t]).wait()
    plsc.subcore_barrier()

    pltpu.sync_copy(acc_sh, z_v)
    w = _wid()
    pltpu.sync_copy(src32_hbm.at[w], fsidx)
    pltpu.sync_copy(dst32_hbm.at[w], fdidx)
    b2 = b2_v[...]

    def zbody(i, carry):
        s = pl.ds(i * L, L)
        z_v[s] = dis_v[s] * (z_v[s] + u_v[s]) + b2
        return carry

    lax.fori_loop(0, NP // L, zbody, 0)

    one = jnp.ones((L,), jnp.float32)

    def dchunk(j, carry):
        for l in range(KD // L):
            cs = pl.ds(l * L, L)
            zs = plsc.load_gather(z_v, [fsidx[j, cs]])
            zd = plsc.load_gather(z_v, [fdidx[j, cs]])
            out_v[j, cs] = one / (one + jnp.exp(-(zs * zd)))
        return carry

    lax.fori_loop(0, NCHF, dchunk, 0)
    pltpu.sync_copy(out_v, out_hbm.at[w])


def _tc_encode_body(degp_ref, x_ref, w1_ref, y_ref, dis_ref):
    deg = degp_ref[0, :] + degp_ref[1, :] + 1.0
    dis = lax.rsqrt(deg)
    xw = jnp.dot(x_ref[...], w1_ref[...], preferred_element_type=jnp.float32)
    y_ref[...] = xw * dis[:, None]
    dis_ref[...] = dis


def _tc_mid_body(accp_ref, y_ref, dis_ref, b1_ref, w2_ref, u_ref):
    dis = dis_ref[...]
    t = (accp_ref[0] + accp_ref[1] + y_ref[...]) * dis[:, None]
    h = jnp.maximum(t + b1_ref[...][None, :], 0.0)
    hw = jnp.sum(h * w2_ref[...][:, 0][None, :], axis=1)
    u_ref[...] = hw * dis


def kernel(x, edge_index, W1, b1, W2, b2):
    base = jnp.arange(EP - E, dtype=jnp.int32)
    padA = N + base % (NP - N)
    padB = N + (base + 80) % (NP - N)
    padC = N + (base + 160) % (NP - N)
    srcA = jnp.concatenate([edge_index[0], padA])
    dstA = jnp.concatenate([edge_index[1], padA])
    srcB = jnp.concatenate([edge_index[0], padB])
    dstB = jnp.concatenate([edge_index[1], padB])
    srcC = jnp.concatenate([edge_index[0], padC])
    dstC = jnp.concatenate([edge_index[1], padC])
    dst_deg = dstA.reshape(NW, NCHD, KD)
    src_ra = srcB.reshape(NW, NCHR, KR)
    dst_ra = dstB.reshape(NW, NCHR, KR)
    src16 = srcC.reshape(NS, NCHS, KS)
    dst16 = dstC.reshape(NS, NCHS, KS)
    src_de = srcA.reshape(NW, NCHF, KD)
    dst_de = dstA.reshape(NW, NCHF, KD)

    xp = jnp.pad(x, ((0, NP - N), (0, 0)))
    zeros_n = jnp.zeros((NP,), jnp.float32)
    zeros_nd = jnp.zeros((NP, D), jnp.float32)
    b2v = jnp.broadcast_to(b2.astype(jnp.float32), (L,))

    degp = _deg_kernel(dst_deg, zeros_n)

    y, dis = pl.pallas_call(
        _tc_encode_body,
        out_shape=[
            jax.ShapeDtypeStruct((NP, D), jnp.float32),
            jax.ShapeDtypeStruct((NP,), jnp.float32),
        ],
    )(degp, xp, W1)

    accp = _row_agg_kernel(src_ra, dst_ra, y, zeros_nd)

    u = pl.pallas_call(
        _tc_mid_body,
        out_shape=jax.ShapeDtypeStruct((NP,), jnp.float32),
    )(accp, y, dis, b1, W2)

    out = _scal_decode_kernel(src16, dst16, src_de, dst_de, u, dis,
                              zeros_n, b2v)
    return out.reshape(EP)[:E]


# speedup vs baseline: 1.1720x; 1.0562x over previous
import functools

import jax
import jax.numpy as jnp
from jax import lax
from jax.experimental import pallas as pl
from jax.experimental.pallas import tpu as pltpu
from jax.experimental.pallas import tpu_sc as plsc

NC = 2
NS = 16
L = 16
NW = NC * NS

N = 10000
E = 320000
D = 128
NP = 10240
SLICE = NP // NS
EP = 327680
ET = EP // NW

KD = 128
NCHD = ET // KD

KR = 80
NCHR = ET // KR
KB = 32
NB = NCHR // KB
RING = 4
ROUNDS = KB // RING

KS = 128
NCHS = EP // NS // KS
SRING = 8
SROUNDS = NCHS // SRING

NCHF = ET // KD

_mesh = plsc.VectorSubcoreMesh(core_axis_name="c", subcore_axis_name="s")
_sc_params = pltpu.CompilerParams(needs_layout_passes=False)


def _wid():
    return lax.axis_index("s") * NC + lax.axis_index("c")


def _tile_slice():
    sid = lax.axis_index("s")
    return pl.ds(sid * SLICE, SLICE)


@functools.partial(
    pl.kernel,
    out_type=jax.ShapeDtypeStruct((NC, NP), jnp.float32),
    mesh=_mesh,
    compiler_params=_sc_params,
    name="deg_sc",
    scratch_types=[
        pltpu.VMEM((NCHD, KD), jnp.int32),
        pltpu.VMEM((KD,), jnp.float32),
        pltpu.SemaphoreType.DMA,
        pltpu.VMEM_SHARED((NP,), jnp.float32),
    ],
)
def _deg_kernel(dst_hbm, zeros_hbm, out_hbm, idx_v, ones_v, sem, acc_sh):
    cid = lax.axis_index("c")
    sl = _tile_slice()
    for l in range(KD // L):
        ones_v[pl.ds(l * L, L)] = jnp.ones((L,), jnp.float32)
    pltpu.sync_copy(zeros_hbm.at[sl], acc_sh.at[sl])
    pltpu.sync_copy(dst_hbm.at[_wid()], idx_v)
    plsc.subcore_barrier()

    def fire(j, carry):
        pltpu.async_copy(ones_v, acc_sh.at[idx_v.at[j]], sem, add=True)
        return carry

    lax.fori_loop(0, NCHD, fire, 0)

    def drain(j, carry):
        pltpu.make_async_copy(ones_v, acc_sh.at[idx_v.at[0]], sem).wait()
        return carry

    lax.fori_loop(0, NCHD, drain, 0)
    plsc.subcore_barrier()
    pltpu.sync_copy(acc_sh.at[sl], out_hbm.at[cid, sl])


@functools.partial(
    pl.kernel,
    out_type=jax.ShapeDtypeStruct((NC, NP, D), jnp.float32),
    mesh=_mesh,
    compiler_params=_sc_params,
    name="rowagg_sc",
    scratch_types=[
        pltpu.VMEM((KB, KR), jnp.int32),
        pltpu.VMEM((KB, KR), jnp.int32),
        pltpu.VMEM((RING * KR, D), jnp.float32),
        [pltpu.SemaphoreType.DMA] * RING,
        [pltpu.SemaphoreType.DMA] * RING,
        pltpu.VMEM_SHARED((NP, D), jnp.float32),
    ],
)
def _row_agg_kernel(src_hbm, dst_hbm, y_hbm, zeros_hbm, out_hbm,
                    sblk, dblk, rowsb, gsem, ssem, acc_sh):
    bufs = [rowsb.at[pl.ds(t * KR, KR)] for t in range(RING)]
    cid = lax.axis_index("c")
    sl = _tile_slice()
    pltpu.sync_copy(zeros_hbm.at[sl], acc_sh.at[sl])
    w = _wid()
    plsc.subcore_barrier()

    for b in range(NB):
        pltpu.sync_copy(src_hbm.at[w, pl.ds(b * KB, KB)], sblk)
        pltpu.sync_copy(dst_hbm.at[w, pl.ds(b * KB, KB)], dblk)

        for t in range(RING):
            pltpu.async_copy(y_hbm.at[sblk.at[t]], bufs[t], gsem[t])

        def round_body(r, carry):
            for t in range(RING):
                j = RING * r + t
                pltpu.make_async_copy(
                    y_hbm.at[sblk.at[j]], bufs[t], gsem[t]).wait()
                pltpu.async_copy(bufs[t], acc_sh.at[dblk.at[j]], ssem[t],
                                 add=True)

            @pl.when(r < ROUNDS - 1)
            def _():
                for t in range(RING):
                    j = RING * r + t
                    pltpu.make_async_copy(
                        bufs[t], acc_sh.at[dblk.at[j]], ssem[t]).wait()
                    pltpu.async_copy(
                        y_hbm.at[sblk.at[j + RING]], bufs[t], gsem[t])

            return carry

        lax.fori_loop(0, ROUNDS, round_body, 0)
        for t in range(RING):
            pltpu.make_async_copy(
                bufs[t], acc_sh.at[dblk.at[KB - RING + t]], ssem[t]).wait()

    plsc.subcore_barrier()
    pltpu.sync_copy(acc_sh.at[sl], out_hbm.at[cid, sl])


@functools.partial(
    pl.kernel,
    out_type=jax.ShapeDtypeStruct((NW, NCHF, KD), jnp.float32),
    mesh=_mesh,
    compiler_params=_sc_params,
    name="scaldec_sc",
    scratch_types=[
        pltpu.VMEM((NCHS, KS), jnp.int32),
        pltpu.VMEM((NCHS, KS), jnp.int32),
        pltpu.VMEM((NCHF, KD), jnp.int32),
        pltpu.VMEM((NCHF, KD), jnp.int32),
        pltpu.VMEM((NP,), jnp.float32),
        pltpu.VMEM((NP,), jnp.float32),
        pltpu.VMEM((NP,), jnp.float32),
        pltpu.VMEM((SRING, KS), jnp.float32),
        pltpu.VMEM((NCHF, KD), jnp.float32),
        pltpu.VMEM((L,), jnp.float32),
        [pltpu.SemaphoreType.DMA] * SRING,
        pltpu.VMEM_SHARED((NP,), jnp.float32),
    ],
)
def _scal_decode_kernel(src16_hbm, dst16_hbm, src32_hbm, dst32_hbm,
                        u_hbm, dis_hbm, zeros_hbm, b2_hbm, out_hbm,
                        sidx, didx, fsidx, fdidx, u_v, dis_v, z_v,
                        ring, out_v, b2_v, ssem, acc_sh):
    sid = lax.axis_index("s")
    sl = _tile_slice()
    pltpu.sync_copy(zeros_hbm.at[sl], acc_sh.at[sl])
    pltpu.sync_copy(src16_hbm.at[sid], sidx)
    pltpu.sync_copy(dst16_hbm.at[sid], didx)
    pltpu.sync_copy(u_hbm, u_v)
    pltpu.sync_copy(dis_hbm, dis_v)
    pltpu.sync_copy(b2_hbm, b2_v)
    plsc.subcore_barrier()

    def srow(r, carry):
        for t in range(SRING):
            j = SRING * r + t

            @pl.when(r > 0)
            def _():
                pltpu.make_async_copy(
                    ring.at[t], acc_sh.at[didx.at[j]], ssem[t]).wait()

            for l in range(KS // L):
                iv = sidx[j, pl.ds(l * L, L)]
                ring[t, pl.ds(l * L, L)] = plsc.load_gather(u_v, [iv])
            pltpu.async_copy(ring.at[t], acc_sh.at[didx.at[j]], ssem[t],
                             add=True)
        return carry

    lax.fori_loop(0, SROUNDS, srow, 0)
    for t in range(SRING):
        pltpu.make_async_copy(
            ring.at[t], acc_sh.at[didx.at[0]], ssem[t]).wait()
    plsc.subcore_barrier()

    pltpu.sync_copy(acc_sh, z_v)
    w = _wid()
    pltpu.sync_copy(src32_hbm.at[w], fsidx)
    pltpu.sync_copy(dst32_hbm.at[w], fdidx)
    b2 = b2_v[...]

    def zbody(i, carry):
        s = pl.ds(i * L, L)
        z_v[s] = dis_v[s] * (z_v[s] + u_v[s]) + b2
        return carry

    lax.fori_loop(0, NP // L, zbody, 0)

    one = jnp.ones((L,), jnp.float32)

    def dchunk(j, carry):
        for l in range(KD // L):
            cs = pl.ds(l * L, L)
            zs = plsc.load_gather(z_v, [fsidx[j, cs]])
            zd = plsc.load_gather(z_v, [fdidx[j, cs]])
            out_v[j, cs] = one / (one + jnp.exp(-(zs * zd)))
        return carry

    lax.fori_loop(0, NCHF, dchunk, 0)
    pltpu.sync_copy(out_v, out_hbm.at[w])


def _tc_encode_body(degp_ref, x_ref, w1_ref, y_ref, dis_ref):
    deg = degp_ref[0, :] + degp_ref[1, :] + 1.0
    dis = lax.rsqrt(deg)
    xw = jnp.dot(x_ref[...], w1_ref[...], preferred_element_type=jnp.float32)
    y_ref[...] = xw * dis[:, None]
    dis_ref[...] = dis


def _tc_mid_body(accp_ref, y_ref, dis_ref, b1_ref, w2_ref, u_ref):
    dis = dis_ref[...]
    t = (accp_ref[0] + accp_ref[1] + y_ref[...]) * dis[:, None]
    h = jnp.maximum(t + b1_ref[...][None, :], 0.0)
    hw = jnp.sum(h * w2_ref[...][:, 0][None, :], axis=1)
    u_ref[...] = hw * dis


def kernel(x, edge_index, W1, b1, W2, b2):
    base = jnp.arange(EP - E, dtype=jnp.int32)
    padA = N + base % (NP - N)
    padB = N + (base + 80) % (NP - N)
    padC = N + (base + 160) % (NP - N)
    srcA = jnp.concatenate([edge_index[0], padA])
    dstA = jnp.concatenate([edge_index[1], padA])
    srcB = jnp.concatenate([edge_index[0], padB])
    dstB = jnp.concatenate([edge_index[1], padB])
    srcC = jnp.concatenate([edge_index[0], padC])
    dstC = jnp.concatenate([edge_index[1], padC])
    dst_deg = dstA.reshape(NW, NCHD, KD)
    src_ra = srcB.reshape(NW, NCHR, KR)
    dst_ra = dstB.reshape(NW, NCHR, KR)
    src16 = srcC.reshape(NS, NCHS, KS)
    dst16 = dstC.reshape(NS, NCHS, KS)
    src_de = srcA.reshape(NW, NCHF, KD)
    dst_de = dstA.reshape(NW, NCHF, KD)

    xp = jnp.pad(x, ((0, NP - N), (0, 0)))
    zeros_n = jnp.zeros((NP,), jnp.float32)
    zeros_nd = jnp.zeros((NP, D), jnp.float32)
    b2v = jnp.broadcast_to(b2.astype(jnp.float32), (L,))

    degp = _deg_kernel(dst_deg, zeros_n)

    y, dis = pl.pallas_call(
        _tc_encode_body,
        out_shape=[
            jax.ShapeDtypeStruct((NP, D), jnp.float32),
            jax.ShapeDtypeStruct((NP,), jnp.float32),
        ],
    )(degp, xp, W1)

    accp = _row_agg_kernel(src_ra, dst_ra, y, zeros_nd)

    u = pl.pallas_call(
        _tc_mid_body,
        out_shape=jax.ShapeDtypeStruct((NP,), jnp.float32),
    )(accp, y, dis, b1, W2)

    out = _scal_decode_kernel(src16, dst16, src_de, dst_de, u, dis,
                              zeros_n, b2v)
    return out.reshape(EP)[:E]


# speedup vs baseline: 1.1841x; 1.0103x over previous
import functools

import jax
import jax.numpy as jnp
from jax import lax
from jax.experimental import pallas as pl
from jax.experimental.pallas import tpu as pltpu
from jax.experimental.pallas import tpu_sc as plsc

NC = 2
NS = 16
L = 16
NW = NC * NS

N = 10000
E = 320000
D = 128
NP = 10240
SLICE = NP // NS
EP = 327680
ET = EP // NW

KD = 128
NCHD = ET // KD

KR = 80
NCHR = ET // KR
KB = 32
NB = NCHR // KB
RING = 4
ROUNDS = KB // RING

KS = 128
NCHS = EP // NS // KS
SRING = 8
SROUNDS = NCHS // SRING

NCHF = ET // KD

_mesh = plsc.VectorSubcoreMesh(core_axis_name="c", subcore_axis_name="s")
_sc_params = pltpu.CompilerParams(needs_layout_passes=False)


def _wid():
    return lax.axis_index("s") * NC + lax.axis_index("c")


def _tile_slice():
    sid = lax.axis_index("s")
    return pl.ds(sid * SLICE, SLICE)


@functools.partial(
    pl.kernel,
    out_type=jax.ShapeDtypeStruct((NC, NP), jnp.float32),
    mesh=_mesh,
    compiler_params=_sc_params,
    name="deg_sc",
    scratch_types=[
        pltpu.VMEM((NCHD, KD), jnp.int32),
        pltpu.VMEM((KD,), jnp.float32),
        pltpu.SemaphoreType.DMA,
        pltpu.VMEM_SHARED((NP,), jnp.float32),
    ],
)
def _deg_kernel(dst_hbm, zeros_hbm, out_hbm, idx_v, ones_v, sem, acc_sh):
    cid = lax.axis_index("c")
    sl = _tile_slice()
    for l in range(KD // L):
        ones_v[pl.ds(l * L, L)] = jnp.ones((L,), jnp.float32)
    pltpu.sync_copy(zeros_hbm.at[sl], acc_sh.at[sl])
    pltpu.sync_copy(dst_hbm.at[_wid()], idx_v)
    plsc.subcore_barrier()

    def fire(j, carry):
        pltpu.async_copy(ones_v, acc_sh.at[idx_v.at[j]], sem, add=True)
        return carry

    lax.fori_loop(0, NCHD, fire, 0)

    def drain(j, carry):
        pltpu.make_async_copy(ones_v, acc_sh.at[idx_v.at[0]], sem).wait()
        return carry

    lax.fori_loop(0, NCHD, drain, 0)
    plsc.subcore_barrier()
    pltpu.sync_copy(acc_sh.at[sl], out_hbm.at[cid, sl])


@functools.partial(
    pl.kernel,
    out_type=jax.ShapeDtypeStruct((NC, NP, D), jnp.float32),
    mesh=_mesh,
    compiler_params=_sc_params,
    name="rowagg_sc",
    scratch_types=[
        pltpu.VMEM((KB, KR), jnp.int32),
        pltpu.VMEM((KB, KR), jnp.int32),
        pltpu.VMEM((RING * KR, D), jnp.float32),
        [pltpu.SemaphoreType.DMA] * RING,
        [pltpu.SemaphoreType.DMA] * RING,
        pltpu.VMEM_SHARED((NP, D), jnp.float32),
    ],
)
def _row_agg_kernel(src_hbm, dst_hbm, y_hbm, zeros_hbm, out_hbm,
                    sblk, dblk, rowsb, gsem, ssem, acc_sh):
    bufs = [rowsb.at[pl.ds(t * KR, KR)] for t in range(RING)]
    cid = lax.axis_index("c")
    sl = _tile_slice()

    @pl.when(cid == 0)
    def _():
        pltpu.sync_copy(y_hbm.at[sl], acc_sh.at[sl])

    @pl.when(cid != 0)
    def _():
        pltpu.sync_copy(zeros_hbm.at[sl], acc_sh.at[sl])

    w = _wid()
    plsc.subcore_barrier()

    for b in range(NB):
        pltpu.sync_copy(src_hbm.at[w, pl.ds(b * KB, KB)], sblk)
        pltpu.sync_copy(dst_hbm.at[w, pl.ds(b * KB, KB)], dblk)

        for t in range(RING):
            pltpu.async_copy(y_hbm.at[sblk.at[t]], bufs[t], gsem[t])

        def round_body(r, carry):
            for t in range(RING):
                j = RING * r + t
                pltpu.make_async_copy(
                    y_hbm.at[sblk.at[j]], bufs[t], gsem[t]).wait()
                pltpu.async_copy(bufs[t], acc_sh.at[dblk.at[j]], ssem[t],
                                 add=True)

            @pl.when(r < ROUNDS - 1)
            def _():
                for t in range(RING):
                    j = RING * r + t
                    pltpu.make_async_copy(
                        bufs[t], acc_sh.at[dblk.at[j]], ssem[t]).wait()
                    pltpu.async_copy(
                        y_hbm.at[sblk.at[j + RING]], bufs[t], gsem[t])

            return carry

        lax.fori_loop(0, ROUNDS, round_body, 0)
        for t in range(RING):
            pltpu.make_async_copy(
                bufs[t], acc_sh.at[dblk.at[KB - RING + t]], ssem[t]).wait()

    plsc.subcore_barrier()
    pltpu.sync_copy(acc_sh.at[sl], out_hbm.at[cid, sl])


@functools.partial(
    pl.kernel,
    out_type=jax.ShapeDtypeStruct((NW, NCHF, KD), jnp.float32),
    mesh=_mesh,
    compiler_params=_sc_params,
    name="scaldec_sc",
    scratch_types=[
        pltpu.VMEM((NCHS, KS), jnp.int32),
        pltpu.VMEM((NCHS, KS), jnp.int32),
        pltpu.VMEM((NCHF, KD), jnp.int32),
        pltpu.VMEM((NCHF, KD), jnp.int32),
        pltpu.VMEM((NP,), jnp.float32),
        pltpu.VMEM((NP,), jnp.float32),
        pltpu.VMEM((NP,), jnp.float32),
        pltpu.VMEM((SRING, KS), jnp.float32),
        pltpu.VMEM((NCHF, KD), jnp.float32),
        pltpu.VMEM((L,), jnp.float32),
        [pltpu.SemaphoreType.DMA] * SRING,
        pltpu.VMEM_SHARED((NP,), jnp.float32),
    ],
)
def _scal_decode_kernel(src16_hbm, dst16_hbm, src32_hbm, dst32_hbm,
                        u_hbm, dis_hbm, b2_hbm, out_hbm,
                        sidx, didx, fsidx, fdidx, u_v, dis_v, z_v,
                        ring, out_v, b2_v, ssem, acc_sh):
    sid = lax.axis_index("s")
    sl = _tile_slice()
    pltpu.sync_copy(u_hbm.at[sl], acc_sh.at[sl])
    pltpu.sync_copy(src16_hbm.at[sid], sidx)
    pltpu.sync_copy(dst16_hbm.at[sid], didx)
    pltpu.sync_copy(u_hbm, u_v)
    pltpu.sync_copy(dis_hbm, dis_v)
    pltpu.sync_copy(b2_hbm, b2_v)
    plsc.subcore_barrier()

    def srow(r, carry):
        for t in range(SRING):
            j = SRING * r + t

            @pl.when(r > 0)
            def _():
                pltpu.make_async_copy(
                    ring.at[t], acc_sh.at[didx.at[j]], ssem[t]).wait()

            for l in range(KS // L):
                iv = sidx[j, pl.ds(l * L, L)]
                ring[t, pl.ds(l * L, L)] = plsc.load_gather(u_v, [iv])
            pltpu.async_copy(ring.at[t], acc_sh.at[didx.at[j]], ssem[t],
                             add=True)
        return carry

    lax.fori_loop(0, SROUNDS, srow, 0)
    for t in range(SRING):
        pltpu.make_async_copy(
            ring.at[t], acc_sh.at[didx.at[0]], ssem[t]).wait()
    plsc.subcore_barrier()

    pltpu.sync_copy(acc_sh, z_v)
    w = _wid()
    pltpu.sync_copy(src32_hbm.at[w], fsidx)
    pltpu.sync_copy(dst32_hbm.at[w], fdidx)
    b2 = b2_v[...]

    def zbody(i, carry):
        s = pl.ds(i * L, L)
        z_v[s] = dis_v[s] * z_v[s] + b2
        return carry

    lax.fori_loop(0, NP // L, zbody, 0)

    one = jnp.ones((L,), jnp.float32)

    def dchunk(j, carry):
        for l in range(KD // L):
            cs = pl.ds(l * L, L)
            zs = plsc.load_gather(z_v, [fsidx[j, cs]])
            zd = plsc.load_gather(z_v, [fdidx[j, cs]])
            out_v[j, cs] = one / (one + jnp.exp(-(zs * zd)))
        return carry

    lax.fori_loop(0, NCHF, dchunk, 0)
    pltpu.sync_copy(out_v, out_hbm.at[w])


def _tc_encode_body(degp_ref, x_ref, w1_ref, y_ref, dis_ref):
    deg = degp_ref[0, :] + degp_ref[1, :] + 1.0
    dis = lax.rsqrt(deg)
    xw = jnp.dot(x_ref[...], w1_ref[...], preferred_element_type=jnp.float32)
    y_ref[...] = xw * dis[:, None]
    dis_ref[...] = dis


def _tc_mid_body(accp_ref, dis_ref, b1_ref, w2_ref, u_ref):
    dis = dis_ref[...]
    t = (accp_ref[0] + accp_ref[1]) * dis[:, None]
    h = jnp.maximum(t + b1_ref[...][None, :], 0.0)
    hw = jnp.sum(h * w2_ref[...][:, 0][None, :], axis=1)
    u_ref[...] = hw * dis


def kernel(x, edge_index, W1, b1, W2, b2):
    base = jnp.arange(EP - E, dtype=jnp.int32)
    padA = N + base % (NP - N)
    padB = N + (base + 80) % (NP - N)
    padC = N + (base + 160) % (NP - N)
    srcA = jnp.concatenate([edge_index[0], padA])
    dstA = jnp.concatenate([edge_index[1], padA])
    srcB = jnp.concatenate([edge_index[0], padB])
    dstB = jnp.concatenate([edge_index[1], padB])
    srcC = jnp.concatenate([edge_index[0], padC])
    dstC = jnp.concatenate([edge_index[1], padC])
    dst_deg = dstA.reshape(NW, NCHD, KD)
    src_ra = srcB.reshape(NW, NCHR, KR)
    dst_ra = dstB.reshape(NW, NCHR, KR)
    src16 = srcC.reshape(NS, NCHS, KS)
    dst16 = dstC.reshape(NS, NCHS, KS)
    src_de = srcA.reshape(NW, NCHF, KD)
    dst_de = dstA.reshape(NW, NCHF, KD)

    xp = jnp.pad(x, ((0, NP - N), (0, 0)))
    zeros_n = jnp.zeros((NP,), jnp.float32)
    zeros_nd = jnp.zeros((NP, D), jnp.float32)
    b2v = jnp.broadcast_to(b2.astype(jnp.float32), (L,))

    degp = _deg_kernel(dst_deg, zeros_n)

    y, dis = pl.pallas_call(
        _tc_encode_body,
        out_shape=[
            jax.ShapeDtypeStruct((NP, D), jnp.float32),
            jax.ShapeDtypeStruct((NP,), jnp.float32),
        ],
    )(degp, xp, W1)

    accp = _row_agg_kernel(src_ra, dst_ra, y, zeros_nd)

    u = pl.pallas_call(
        _tc_mid_body,
        out_shape=jax.ShapeDtypeStruct((NP,), jnp.float32),
    )(accp, dis, b1, W2)

    out = _scal_decode_kernel(src16, dst16, src_de, dst_de, u, dis, b2v)
    return out.reshape(EP)[:E]


# speedup vs baseline: 1.1848x; 1.0006x over previous
import functools

import jax
import jax.numpy as jnp
from jax import lax
from jax.experimental import pallas as pl
from jax.experimental.pallas import tpu as pltpu
from jax.experimental.pallas import tpu_sc as plsc

NC = 2
NS = 16
L = 16
NW = NC * NS

N = 10000
E = 320000
D = 128
NP = 10240
SLICE = NP // NS
EP = 327680
ET = EP // NW

KD = 128
NCHD = ET // KD

KR = 80
NCHR = ET // KR
KB = 16
NB = NCHR // KB
RING = 4
ROUNDS = KB // RING

KS = 128
NCHS = EP // NS // KS
SRING = 8
SROUNDS = NCHS // SRING

NCHF = ET // KD

_mesh = plsc.VectorSubcoreMesh(core_axis_name="c", subcore_axis_name="s")
_sc_params = pltpu.CompilerParams(needs_layout_passes=False)


def _wid():
    return lax.axis_index("s") * NC + lax.axis_index("c")


def _tile_slice():
    sid = lax.axis_index("s")
    return pl.ds(sid * SLICE, SLICE)


@functools.partial(
    pl.kernel,
    out_type=jax.ShapeDtypeStruct((NC, NP), jnp.float32),
    mesh=_mesh,
    compiler_params=_sc_params,
    name="deg_sc",
    scratch_types=[
        pltpu.VMEM((NCHD, KD), jnp.int32),
        pltpu.VMEM((KD,), jnp.float32),
        pltpu.SemaphoreType.DMA,
        pltpu.VMEM_SHARED((NP,), jnp.float32),
    ],
)
def _deg_kernel(dst_hbm, zeros_hbm, out_hbm, idx_v, ones_v, sem, acc_sh):
    cid = lax.axis_index("c")
    sl = _tile_slice()
    for l in range(KD // L):
        ones_v[pl.ds(l * L, L)] = jnp.ones((L,), jnp.float32)
    pltpu.sync_copy(zeros_hbm.at[sl], acc_sh.at[sl])
    pltpu.sync_copy(dst_hbm.at[_wid()], idx_v)
    plsc.subcore_barrier()

    def fire(j, carry):
        pltpu.async_copy(ones_v, acc_sh.at[idx_v.at[j]], sem, add=True)
        return carry

    lax.fori_loop(0, NCHD, fire, 0)

    def drain(j, carry):
        pltpu.make_async_copy(ones_v, acc_sh.at[idx_v.at[0]], sem).wait()
        return carry

    lax.fori_loop(0, NCHD, drain, 0)
    plsc.subcore_barrier()
    pltpu.sync_copy(acc_sh.at[sl], out_hbm.at[cid, sl])


@functools.partial(
    pl.kernel,
    out_type=jax.ShapeDtypeStruct((NC, NP, D), jnp.float32),
    mesh=_mesh,
    compiler_params=_sc_params,
    name="rowagg_sc",
    scratch_types=[
        pltpu.VMEM((KB, KR), jnp.int32),
        pltpu.VMEM((KB, KR), jnp.int32),
        pltpu.VMEM((KB, KR), jnp.int32),
        pltpu.VMEM((KB, KR), jnp.int32),
        pltpu.VMEM((RING * KR, D), jnp.float32),
        [pltpu.SemaphoreType.DMA] * RING,
        [pltpu.SemaphoreType.DMA] * RING,
        pltpu.SemaphoreType.DMA,
        pltpu.VMEM_SHARED((NP, D), jnp.float32),
    ],
)
def _row_agg_kernel(src_hbm, dst_hbm, y_hbm, zeros_hbm, out_hbm,
                    sblkA, dblkA, sblkB, dblkB, rowsb, gsem, ssem, isem,
                    acc_sh):
    bufs = [rowsb.at[pl.ds(t * KR, KR)] for t in range(RING)]
    cid = lax.axis_index("c")
    sl = _tile_slice()

    @pl.when(cid == 0)
    def _():
        pltpu.sync_copy(y_hbm.at[sl], acc_sh.at[sl])

    @pl.when(cid != 0)
    def _():
        pltpu.sync_copy(zeros_hbm.at[sl], acc_sh.at[sl])

    w = _wid()
    plsc.subcore_barrier()

    pltpu.sync_copy(src_hbm.at[w, pl.ds(0, KB)], sblkA)
    pltpu.sync_copy(dst_hbm.at[w, pl.ds(0, KB)], dblkA)
    for b in range(NB):
        sblk, dblk = (sblkA, dblkA) if b % 2 == 0 else (sblkB, dblkB)
        nsblk, ndblk = (sblkB, dblkB) if b % 2 == 0 else (sblkA, dblkA)
        if b + 1 < NB:
            pltpu.async_copy(src_hbm.at[w, pl.ds((b + 1) * KB, KB)],
                             nsblk, isem)
            pltpu.async_copy(dst_hbm.at[w, pl.ds((b + 1) * KB, KB)],
                             ndblk, isem)

        for t in range(RING):
            pltpu.async_copy(y_hbm.at[sblk.at[t]], bufs[t], gsem[t])

        def round_body(r, carry, sblk=sblk, dblk=dblk):
            for t in range(RING):
                j = RING * r + t
                pltpu.make_async_copy(
                    y_hbm.at[sblk.at[j]], bufs[t], gsem[t]).wait()
                pltpu.async_copy(bufs[t], acc_sh.at[dblk.at[j]], ssem[t],
                                 add=True)

            @pl.when(r < ROUNDS - 1)
            def _():
                for t in range(RING):
                    j = RING * r + t
                    pltpu.make_async_copy(
                        bufs[t], acc_sh.at[dblk.at[j]], ssem[t]).wait()
                    pltpu.async_copy(
                        y_hbm.at[sblk.at[j + RING]], bufs[t], gsem[t])

            return carry

        lax.fori_loop(0, ROUNDS, round_body, 0)
        for t in range(RING):
            pltpu.make_async_copy(
                bufs[t], acc_sh.at[dblk.at[KB - RING + t]], ssem[t]).wait()
        if b + 1 < NB:
            pltpu.make_async_copy(
                src_hbm.at[w, pl.ds((b + 1) * KB, KB)], nsblk, isem).wait()
            pltpu.make_async_copy(
                dst_hbm.at[w, pl.ds((b + 1) * KB, KB)], ndblk, isem).wait()

    plsc.subcore_barrier()
    pltpu.sync_copy(acc_sh.at[sl], out_hbm.at[cid, sl])


@functools.partial(
    pl.kernel,
    out_type=jax.ShapeDtypeStruct((NW, NCHF, KD), jnp.float32),
    mesh=_mesh,
    compiler_params=_sc_params,
    name="scaldec_sc",
    scratch_types=[
        pltpu.VMEM((NCHS, KS), jnp.int32),
        pltpu.VMEM((NCHS, KS), jnp.int32),
        pltpu.VMEM((NCHF, KD), jnp.int32),
        pltpu.VMEM((NCHF, KD), jnp.int32),
        pltpu.VMEM((NP,), jnp.float32),
        pltpu.VMEM((NP,), jnp.float32),
        pltpu.VMEM((NP,), jnp.float32),
        pltpu.VMEM((SRING, KS), jnp.float32),
        pltpu.VMEM((NCHF, KD), jnp.float32),
        pltpu.VMEM((L,), jnp.float32),
        [pltpu.SemaphoreType.DMA] * SRING,
        pltpu.VMEM_SHARED((NP,), jnp.float32),
    ],
)
def _scal_decode_kernel(src16_hbm, dst16_hbm, src32_hbm, dst32_hbm,
                        u_hbm, dis_hbm, b2_hbm, out_hbm,
                        sidx, didx, fsidx, fdidx, u_v, dis_v, z_v,
                        ring, out_v, b2_v, ssem, acc_sh):
    sid = lax.axis_index("s")
    sl = _tile_slice()
    pltpu.sync_copy(u_hbm.at[sl], acc_sh.at[sl])
    pltpu.sync_copy(src16_hbm.at[sid], sidx)
    pltpu.sync_copy(dst16_hbm.at[sid], didx)
    pltpu.sync_copy(u_hbm, u_v)
    pltpu.sync_copy(dis_hbm, dis_v)
    pltpu.sync_copy(b2_hbm, b2_v)
    plsc.subcore_barrier()

    def srow(r, carry):
        for t in range(SRING):
            j = SRING * r + t

            @pl.when(r > 0)
            def _():
                pltpu.make_async_copy(
                    ring.at[t], acc_sh.at[didx.at[j]], ssem[t]).wait()

            for l in range(KS // L):
                iv = sidx[j, pl.ds(l * L, L)]
                ring[t, pl.ds(l * L, L)] = plsc.load_gather(u_v, [iv])
            pltpu.async_copy(ring.at[t], acc_sh.at[didx.at[j]], ssem[t],
                             add=True)
        return carry

    lax.fori_loop(0, SROUNDS, srow, 0)
    for t in range(SRING):
        pltpu.make_async_copy(
            ring.at[t], acc_sh.at[didx.at[0]], ssem[t]).wait()
    plsc.subcore_barrier()

    pltpu.sync_copy(acc_sh, z_v)
    w = _wid()
    pltpu.sync_copy(src32_hbm.at[w], fsidx)
    pltpu.sync_copy(dst32_hbm.at[w], fdidx)
    b2 = b2_v[...]

    def zbody(i, carry):
        s = pl.ds(i * L, L)
        z_v[s] = dis_v[s] * z_v[s] + b2
        return carry

    lax.fori_loop(0, NP // L, zbody, 0)

    one = jnp.ones((L,), jnp.float32)

    def dchunk(j, carry):
        for l in range(KD // L):
            cs = pl.ds(l * L, L)
            zs = plsc.load_gather(z_v, [fsidx[j, cs]])
            zd = plsc.load_gather(z_v, [fdidx[j, cs]])
            out_v[j, cs] = one / (one + jnp.exp(-(zs * zd)))
        return carry

    lax.fori_loop(0, NCHF, dchunk, 0)
    pltpu.sync_copy(out_v, out_hbm.at[w])


def _tc_encode_body(degp_ref, x_ref, w1_ref, y_ref, dis_ref):
    deg = degp_ref[0, :] + degp_ref[1, :] + 1.0
    dis = lax.rsqrt(deg)
    xw = jnp.dot(x_ref[...], w1_ref[...], preferred_element_type=jnp.float32)
    y_ref[...] = xw * dis[:, None]
    dis_ref[...] = dis


def _tc_mid_body(accp_ref, dis_ref, b1_ref, w2_ref, u_ref):
    dis = dis_ref[...]
    t = (accp_ref[0] + accp_ref[1]) * dis[:, None]
    h = jnp.maximum(t + b1_ref[...][None, :], 0.0)
    hw = jnp.sum(h * w2_ref[...][:, 0][None, :], axis=1)
    u_ref[...] = hw * dis


def kernel(x, edge_index, W1, b1, W2, b2):
    base = jnp.arange(EP - E, dtype=jnp.int32)
    padA = N + base % (NP - N)
    padB = N + (base + 80) % (NP - N)
    padC = N + (base + 160) % (NP - N)
    srcA = jnp.concatenate([edge_index[0], padA])
    dstA = jnp.concatenate([edge_index[1], padA])
    srcB = jnp.concatenate([edge_index[0], padB])
    dstB = jnp.concatenate([edge_index[1], padB])
    srcC = jnp.concatenate([edge_index[0], padC])
    dstC = jnp.concatenate([edge_index[1], padC])
    dst_deg = dstA.reshape(NW, NCHD, KD)
    src_ra = srcB.reshape(NW, NCHR, KR)
    dst_ra = dstB.reshape(NW, NCHR, KR)
    src16 = srcC.reshape(NS, NCHS, KS)
    dst16 = dstC.reshape(NS, NCHS, KS)
    src_de = srcA.reshape(NW, NCHF, KD)
    dst_de = dstA.reshape(NW, NCHF, KD)

    xp = jnp.pad(x, ((0, NP - N), (0, 0)))
    zeros_n = jnp.zeros((NP,), jnp.float32)
    zeros_nd = jnp.zeros((NP, D), jnp.float32)
    b2v = jnp.broadcast_to(b2.astype(jnp.float32), (L,))

    degp = _deg_kernel(dst_deg, zeros_n)

    y, dis = pl.pallas_call(
        _tc_encode_body,
        out_shape=[
            jax.ShapeDtypeStruct((NP, D), jnp.float32),
            jax.ShapeDtypeStruct((NP,), jnp.float32),
        ],
    )(degp, xp, W1)

    accp = _row_agg_kernel(src_ra, dst_ra, y, zeros_nd)

    u = pl.pallas_call(
        _tc_mid_body,
        out_shape=jax.ShapeDtypeStruct((NP,), jnp.float32),
    )(accp, dis, b1, W2)

    out = _scal_decode_kernel(src16, dst16, src_de, dst_de, u, dis, b2v)
    return out.reshape(EP)[:E]
